# baseline (speedup 1.0000x reference)
import functools

import jax
import jax.numpy as jnp
from jax import lax
from jax.experimental import pallas as pl
from jax.experimental.pallas import tpu as pltpu
from jax.experimental.pallas import tpu_sc as plsc

N_ROWS = 64
N_COLS = 8192
KTH = 256
NUM_CORES = 2
NUM_SUBCORES = 16
NW = NUM_CORES * NUM_SUBCORES
ROWS_PER_W = N_ROWS // NW
L = 16
U = 4
UL = U * L
TOP_I = -(2 ** 31)


def _sc_kthvalue(x_bits):
    mesh = plsc.VectorSubcoreMesh(
        core_axis_name="c", subcore_axis_name="s",
        num_cores=NUM_CORES, num_subcores=NUM_SUBCORES)

    @functools.partial(
        pl.kernel,
        out_type=(jax.ShapeDtypeStruct((NW, L), jnp.int32),
                  jax.ShapeDtypeStruct((NW, L), jnp.int32)),
        mesh=mesh,
        compiler_params=pltpu.CompilerParams(needs_layout_passes=False),
        scratch_types=[
            pltpu.VMEM((N_COLS,), jnp.int32),
            pltpu.VMEM((N_COLS,), jnp.int32),
            pltpu.VMEM((N_COLS,), jnp.int32),
            pltpu.VMEM((L,), jnp.int32),
            pltpu.VMEM((L,), jnp.int32),
        ],
    )
    def body(x_hbm, vout_hbm, iout_hbm, kbuf0, kbuf1, ibuf, vstage, istage):
        wid = lax.axis_index("s") * NUM_CORES + lax.axis_index("c")
        io = lax.iota(jnp.int32, L)
        one = jnp.int32(1)
        zero = jnp.int32(0)
        top = jnp.int32(TOP_I)
        zvec = jnp.zeros((L,), jnp.int32)

        def popc(mask):
            return plsc.all_reduce_population_count(mask)

        def scat2(kref, kv, iv, keep, off):
            ki = jnp.where(keep, one, zero)
            csum = plsc.cumsum(ki)
            tgt = off + jnp.maximum(csum - 1, 0)
            plsc.store_scatter(kref, [tgt], kv, mask=keep)
            plsc.store_scatter(ibuf, [tgt], iv, mask=keep)
            return off + popc(keep)

        kbufs = (kbuf0, kbuf1)
        for row in range(ROWS_PER_W):
            pltpu.sync_copy(x_hbm.at[wid * ROWS_PER_W + row], kbufs[row])

        res_v = zvec
        res_i = zvec

        for row in range(ROWS_PER_W):
            krow = kbufs[row]

            def pass_a(j, cv):
                base = j * UL
                for u in range(U):
                    b = krow[pl.ds(base + u * L, L)]
                    neg = jnp.logical_and(b < 0, b != top)
                    cv = cv + popc(neg)
                return cv

            c0 = lax.fori_loop(0, N_COLS // UL, pass_a, zvec)[0]

            r0 = jnp.int32(KTH)
            glow0 = r0 <= c0
            selb0 = jnp.logical_not(glow0)
            r0 = jnp.where(glow0, r0, r0 - c0)
            n_after0 = jnp.where(glow0, c0, jnp.int32(N_COLS) - c0)

            def pass_b(j, carry):
                off, cv = carry
                base = j * UL
                for u in range(U):
                    bs = base + u * L
                    b = krow[pl.ds(bs, L)]
                    b = jnp.where(b == top, zero, b)
                    m = lax.shift_right_arithmetic(b, 31)
                    key = lax.bitwise_xor(b, lax.bitwise_or(m, top))
                    keep = (key >= 0) != selb0
                    b30 = lax.bitwise_and(lax.shift_right_logical(key, 30),
                                          one)
                    nxt0 = jnp.logical_and(keep, b30 == 0)
                    cv = cv + popc(nxt0)
                    off = scat2(krow, key, io + bs, keep, off)
                return off, cv

            _, cvb = lax.fori_loop(0, N_COLS // UL, pass_b, (zvec, zvec))
            c1 = cvb[0]

            glow = r0 <= c1
            sel1 = jnp.where(glow, zero, one)
            r1a = jnp.where(glow, r0, r0 - c1)
            n_after1 = jnp.where(glow, c1, n_after0 - c1)
            done1 = n_after0 <= 1
            pb1 = jnp.where(done1, jnp.int32(-1), jnp.int32(30))
            r1 = jnp.where(done1, r0, r1a)

            def level_cond(st):
                pb, _, _, _, _ = st
                return pb >= 0

            def level_body(st):
                pb, sel, n, n_after, r = st
                cb = jnp.maximum(pb - 1, zero)
                nit = lax.div(n + jnp.int32(UL - 1), jnp.int32(UL))

                def chunk(j, carry):
                    off, cv = carry
                    base = j * UL
                    for u in range(U):
                        bs = base + u * L
                        kv = krow[pl.ds(bs, L)]
                        iv = ibuf[pl.ds(bs, L)]
                        ok = (io + bs) < n
                        bitv = lax.bitwise_and(
                            lax.shift_right_logical(kv, pb), one)
                        keep = jnp.logical_and(bitv == sel, ok)
                        nb = lax.bitwise_and(
                            lax.shift_right_logical(kv, cb), one)
                        nxt0 = jnp.logical_and(keep, nb == 0)
                        cv = cv + popc(nxt0)
                        off = scat2(krow, kv, iv, keep, off)
                    return off, cv

                _, cv = lax.fori_loop(0, nit, chunk, (zvec, zvec))
                c = cv[0]
                n_new = n_after
                done = jnp.logical_or(pb == 0, n_new <= 1)
                gl = r <= c
                sel_n = jnp.where(gl, zero, one)
                r_n = jnp.where(done, r, jnp.where(gl, r, r - c))
                n_after_n = jnp.where(gl, c, n_new - c)
                pb_n = jnp.where(done, jnp.int32(-1), pb - 1)
                return (pb_n, sel_n, n_new, n_after_n, r_n)

            st = (pb1, sel1, n_after0, n_after1, r1)
            _, _, _, _, r_fin = lax.while_loop(level_cond, level_body, st)

            pos = jnp.full((L,), r_fin - 1, jnp.int32)
            kv_ans = plsc.load_gather(krow, [pos])
            iv_ans = plsc.load_gather(ibuf, [pos])
            lane = io == row
            res_v = jnp.where(lane, kv_ans, res_v)
            res_i = jnp.where(lane, iv_ans, res_i)

        inv = jnp.where(res_v < 0,
                        lax.bitwise_xor(res_v, top),
                        lax.bitwise_xor(res_v, jnp.int32(-1)))
        vstage[...] = inv
        istage[...] = res_i
        pltpu.sync_copy(vstage, vout_hbm.at[wid])
        pltpu.sync_copy(istage, iout_hbm.at[wid])

    return body(x_bits)


def kernel(x):
    xb = lax.bitcast_convert_type(x, jnp.int32)
    vbits, inds = _sc_kthvalue(xb)
    values = lax.bitcast_convert_type(
        vbits[:, :ROWS_PER_W].reshape(N_ROWS), jnp.float32)
    indices = inds[:, :ROWS_PER_W].reshape(N_ROWS)
    return values, indices.astype(jnp.int64)


# speedup vs baseline: 1.6108x; 1.6108x over previous
import functools

import jax
import jax.numpy as jnp
from jax import lax
from jax.experimental import pallas as pl
from jax.experimental.pallas import tpu as pltpu
from jax.experimental.pallas import tpu_sc as plsc

N_ROWS = 64
N_COLS = 8192
KTH = 256
NUM_CORES = 2
NUM_SUBCORES = 16
NW = NUM_CORES * NUM_SUBCORES
ROWS_PER_W = N_ROWS // NW
L = 16
U = 4
UL = U * L
NBINS = 256
TOP_I = -(2 ** 31)


def _sc_kthvalue(x_bits):
    mesh = plsc.VectorSubcoreMesh(
        core_axis_name="c", subcore_axis_name="s",
        num_cores=NUM_CORES, num_subcores=NUM_SUBCORES)

    @functools.partial(
        pl.kernel,
        out_type=(jax.ShapeDtypeStruct((NW, L), jnp.int32),
                  jax.ShapeDtypeStruct((NW, L), jnp.int32)),
        mesh=mesh,
        compiler_params=pltpu.CompilerParams(needs_layout_passes=False),
        scratch_types=[
            pltpu.VMEM((N_COLS,), jnp.int32),
            pltpu.VMEM((N_COLS,), jnp.int32),
            pltpu.VMEM((NBINS,), jnp.int32),
            pltpu.VMEM((L,), jnp.int32),
            pltpu.VMEM((L,), jnp.int32),
            pltpu.VMEM((L,), jnp.int32),
        ],
    )
    def body(x_hbm, vout_hbm, iout_hbm, kbuf0, kbuf1, hist, bfly, vstage,
             istage):
        wid = lax.axis_index("s") * NUM_CORES + lax.axis_index("c")
        io = lax.iota(jnp.int32, L)
        perms = tuple(lax.bitwise_xor(io, jnp.int32(1 << p))
                      for p in range(3, -1, -1))
        one = jnp.int32(1)
        zero = jnp.int32(0)
        top = jnp.int32(TOP_I)
        zvec = jnp.zeros((L,), jnp.int32)
        ones_v = jnp.full((L,), 1, jnp.int32)

        def popc(mask):
            return plsc.all_reduce_population_count(mask)

        def lane_sum(v):
            for p in perms:
                bfly[...] = v
                v = v + plsc.load_gather(bfly, [p])
            return v

        def hist_zero():
            for j in range(NBINS // L):
                hist[pl.ds(j * L, L)] = zvec

        def hist_scan(r):
            run = zero
            less = zvec
            for j in range(NBINS // L):
                h = hist[pl.ds(j * L, L)]
                csg = plsc.cumsum(h) + run
                hist[pl.ds(j * L, L)] = csg
                run = csg[15]
                less = less + popc(csg < r)
            b = less[0]
            bm1 = jnp.maximum(b - 1, zero)
            prev = plsc.load_gather(hist, [jnp.full((L,), bm1, jnp.int32)])
            cbelow = jnp.where(b == 0, zero, prev[0])
            return b, cbelow

        kbufs = (kbuf0, kbuf1)
        for row in range(ROWS_PER_W):
            pltpu.sync_copy(x_hbm.at[wid * ROWS_PER_W + row], kbufs[row])

        res_v = zvec
        res_i = zvec

        for row in range(ROWS_PER_W):
            krow = kbufs[row]

            hist_zero()

            def pass_h1(j, _):
                base = j * UL
                for u in range(U):
                    bs = base + u * L
                    b = krow[pl.ds(bs, L)]
                    b = jnp.where(b == top, zero, b)
                    m = lax.shift_right_arithmetic(b, 31)
                    key = lax.bitwise_xor(b, lax.bitwise_or(m, top))
                    krow[pl.ds(bs, L)] = key
                    bin1 = lax.shift_right_logical(key, 24)
                    plsc.addupdate_scatter(hist, [bin1], ones_v)
                return 0

            lax.fori_loop(0, N_COLS // UL, pass_h1, 0)
            r = jnp.int32(KTH)
            b1, cb1 = hist_scan(r)
            r = r - cb1

            prefix = b1
            for lvl in range(1, 4):
                hist_zero()
                hi_shift = 32 - 8 * lvl
                lo_shift = 24 - 8 * lvl

                def pass_h(j, _, hi_shift=hi_shift, lo_shift=lo_shift,
                           prefix=prefix):
                    base = j * UL
                    for u in range(U):
                        bs = base + u * L
                        key = krow[pl.ds(bs, L)]
                        m = lax.shift_right_logical(key, hi_shift) == prefix
                        binv = lax.bitwise_and(
                            lax.shift_right_logical(key, lo_shift),
                            jnp.int32(0xFF))
                        plsc.addupdate_scatter(hist, [binv], ones_v, mask=m)
                    return 0

                lax.fori_loop(0, N_COLS // UL, pass_h, 0)
                bl, cbl = hist_scan(r)
                r = r - cbl
                prefix = lax.bitwise_or(
                    lax.shift_left(prefix, jnp.int32(8)), bl)

            v_ans = prefix

            def pass_i(j, carry):
                cnt, pos = carry
                base = j * UL
                for u in range(U):
                    bs = base + u * L
                    key = krow[pl.ds(bs, L)]
                    match = key == v_ans
                    mi = jnp.where(match, one, zero)
                    csg = plsc.cumsum(mi) + cnt
                    hit = jnp.logical_and(match, csg == r)
                    pos = pos + jnp.where(hit, io + bs, zero)
                    cnt = cnt + popc(match)
                return cnt, pos

            _, pos_acc = lax.fori_loop(0, N_COLS // UL, pass_i,
                                       (zvec, zvec))
            pos = lane_sum(pos_acc)

            lane = io == row
            res_v = jnp.where(lane, v_ans, res_v)
            res_i = jnp.where(lane, pos, res_i)

        inv = jnp.where(res_v < 0,
                        lax.bitwise_xor(res_v, top),
                        lax.bitwise_xor(res_v, jnp.int32(-1)))
        vstage[...] = inv
        istage[...] = res_i
        pltpu.sync_copy(vstage, vout_hbm.at[wid])
        pltpu.sync_copy(istage, iout_hbm.at[wid])

    return body(x_bits)


def kernel(x):
    xb = lax.bitcast_convert_type(x, jnp.int32)
    vbits, inds = _sc_kthvalue(xb)
    values = lax.bitcast_convert_type(
        vbits[:, :ROWS_PER_W].reshape(N_ROWS), jnp.float32)
    indices = inds[:, :ROWS_PER_W].reshape(N_ROWS)
    return values, indices.astype(jnp.int64)


# speedup vs baseline: 2.8437x; 1.7654x over previous
import functools

import jax
import jax.numpy as jnp
from jax import lax
from jax.experimental import pallas as pl
from jax.experimental.pallas import tpu as pltpu
from jax.experimental.pallas import tpu_sc as plsc

N_ROWS = 64
N_COLS = 8192
KTH = 256
NUM_CORES = 2
NUM_SUBCORES = 16
NW = NUM_CORES * NUM_SUBCORES
ROWS_PER_W = N_ROWS // NW
L = 16
U = 4
UL = U * L
NBINS = 256
TOP_I = -(2 ** 31)


def _sc_kthvalue(x_bits):
    mesh = plsc.VectorSubcoreMesh(
        core_axis_name="c", subcore_axis_name="s",
        num_cores=NUM_CORES, num_subcores=NUM_SUBCORES)

    @functools.partial(
        pl.kernel,
        out_type=(jax.ShapeDtypeStruct((NW, L), jnp.int32),
                  jax.ShapeDtypeStruct((NW, L), jnp.int32)),
        mesh=mesh,
        compiler_params=pltpu.CompilerParams(needs_layout_passes=False),
        scratch_types=[
            pltpu.VMEM((N_COLS,), jnp.int32),
            pltpu.VMEM((N_COLS,), jnp.int32),
            pltpu.VMEM((NBINS,), jnp.int32),
            pltpu.VMEM((L,), jnp.int32),
            pltpu.VMEM((L,), jnp.int32),
            pltpu.VMEM((L,), jnp.int32),
        ],
    )
    def body(x_hbm, vout_hbm, iout_hbm, kbuf0, kbuf1, hist, bfly, vstage,
             istage):
        wid = lax.axis_index("s") * NUM_CORES + lax.axis_index("c")
        io = lax.iota(jnp.int32, L)
        perms = tuple(lax.bitwise_xor(io, jnp.int32(1 << p))
                      for p in range(3, -1, -1))
        one = jnp.int32(1)
        zero = jnp.int32(0)
        top = jnp.int32(TOP_I)
        zvec = jnp.zeros((L,), jnp.int32)
        ones_v = jnp.full((L,), 1, jnp.int32)

        def popc(mask):
            return plsc.all_reduce_population_count(mask)

        def lane_sum(v):
            for p in perms:
                bfly[...] = v
                v = v + plsc.load_gather(bfly, [p])
            return v

        def hist_zero():
            for j in range(NBINS // L):
                hist[pl.ds(j * L, L)] = zvec

        def hist_scan(r):
            run = zero
            less = zvec
            for j in range(NBINS // L):
                h = hist[pl.ds(j * L, L)]
                csg = plsc.cumsum(h) + run
                hist[pl.ds(j * L, L)] = csg
                run = csg[15]
                less = less + popc(csg < r)
            b = less[0]
            bm1 = jnp.maximum(b - 1, zero)
            prev = plsc.load_gather(hist, [jnp.full((L,), bm1, jnp.int32)])
            cbelow = jnp.where(b == 0, zero, prev[0])
            return b, cbelow

        kbufs = (kbuf0, kbuf1)
        for row in range(ROWS_PER_W):
            pltpu.sync_copy(x_hbm.at[wid * ROWS_PER_W + row], kbufs[row])

        res_v = zvec
        res_i = zvec

        for row in range(ROWS_PER_W):
            krow = kbufs[row]

            hist_zero()

            @plsc.parallel_loop(0, N_COLS // L, unroll=U)
            def pass_h1(j):
                bs = j * L
                b = krow[pl.ds(bs, L)]
                b = jnp.where(b == top, zero, b)
                m = lax.shift_right_arithmetic(b, 31)
                key = lax.bitwise_xor(b, lax.bitwise_or(m, top))
                krow[pl.ds(bs, L)] = key
                bin1 = lax.shift_right_logical(key, 24)
                plsc.addupdate_scatter(hist, [bin1], ones_v)
            r = jnp.int32(KTH)
            b1, cb1 = hist_scan(r)
            r = r - cb1

            prefix = b1
            for lvl in range(1, 4):
                hist_zero()
                hi_shift = 32 - 8 * lvl
                lo_shift = 24 - 8 * lvl

                @plsc.parallel_loop(0, N_COLS // L, unroll=U)
                def pass_h(j, hi_shift=hi_shift, lo_shift=lo_shift,
                           prefix=prefix):
                    bs = j * L
                    key = krow[pl.ds(bs, L)]
                    m = lax.shift_right_logical(key, hi_shift) == prefix
                    binv = lax.bitwise_and(
                        lax.shift_right_logical(key, lo_shift),
                        jnp.int32(0xFF))
                    plsc.addupdate_scatter(hist, [binv], ones_v, mask=m)
                bl, cbl = hist_scan(r)
                r = r - cbl
                prefix = lax.bitwise_or(
                    lax.shift_left(prefix, jnp.int32(8)), bl)

            v_ans = prefix

            @plsc.parallel_loop(0, N_COLS // L, unroll=U,
                                carry=(zvec, zvec))
            def pass_i(j, carry):
                cnt, pos = carry
                bs = j * L
                key = krow[pl.ds(bs, L)]
                match = key == v_ans
                mi = jnp.where(match, one, zero)
                csg = plsc.cumsum(mi) + cnt
                hit = jnp.logical_and(match, csg == r)
                pos = pos + jnp.where(hit, io + bs, zero)
                cnt = cnt + popc(match)
                return cnt, pos

            _, pos_acc = pass_i
            pos = lane_sum(pos_acc)

            lane = io == row
            res_v = jnp.where(lane, v_ans, res_v)
            res_i = jnp.where(lane, pos, res_i)

        inv = jnp.where(res_v < 0,
                        lax.bitwise_xor(res_v, top),
                        lax.bitwise_xor(res_v, jnp.int32(-1)))
        vstage[...] = inv
        istage[...] = res_i
        pltpu.sync_copy(vstage, vout_hbm.at[wid])
        pltpu.sync_copy(istage, iout_hbm.at[wid])

    return body(x_bits)


def kernel(x):
    xb = lax.bitcast_convert_type(x, jnp.int32)
    vbits, inds = _sc_kthvalue(xb)
    values = lax.bitcast_convert_type(
        vbits[:, :ROWS_PER_W].reshape(N_ROWS), jnp.float32)
    indices = inds[:, :ROWS_PER_W].reshape(N_ROWS)
    return values, indices.astype(jnp.int64)


# speedup vs baseline: 2.8459x; 1.0008x over previous
import functools

import jax
import jax.numpy as jnp
from jax import lax
from jax.experimental import pallas as pl
from jax.experimental.pallas import tpu as pltpu
from jax.experimental.pallas import tpu_sc as plsc

N_ROWS = 64
N_COLS = 8192
KTH = 256
NUM_CORES = 2
NUM_SUBCORES = 16
NW = NUM_CORES * NUM_SUBCORES
ROWS_PER_W = N_ROWS // NW
L = 16
U = 8
UL = U * L
NBINS = 256
TOP_I = -(2 ** 31)


def _sc_kthvalue(x_bits):
    mesh = plsc.VectorSubcoreMesh(
        core_axis_name="c", subcore_axis_name="s",
        num_cores=NUM_CORES, num_subcores=NUM_SUBCORES)

    @functools.partial(
        pl.kernel,
        out_type=(jax.ShapeDtypeStruct((NW, L), jnp.int32),
                  jax.ShapeDtypeStruct((NW, L), jnp.int32)),
        mesh=mesh,
        compiler_params=pltpu.CompilerParams(needs_layout_passes=False),
        scratch_types=[
            pltpu.VMEM((N_COLS,), jnp.int32),
            pltpu.VMEM((N_COLS,), jnp.int32),
            pltpu.VMEM((NBINS,), jnp.int32),
            pltpu.VMEM((L,), jnp.int32),
            pltpu.VMEM((L,), jnp.int32),
            pltpu.VMEM((L,), jnp.int32),
        ],
    )
    def body(x_hbm, vout_hbm, iout_hbm, kbuf0, kbuf1, hist, bfly, vstage,
             istage):
        wid = lax.axis_index("s") * NUM_CORES + lax.axis_index("c")
        io = lax.iota(jnp.int32, L)
        perms = tuple(lax.bitwise_xor(io, jnp.int32(1 << p))
                      for p in range(3, -1, -1))
        one = jnp.int32(1)
        zero = jnp.int32(0)
        top = jnp.int32(TOP_I)
        zvec = jnp.zeros((L,), jnp.int32)
        ones_v = jnp.full((L,), 1, jnp.int32)

        def popc(mask):
            return plsc.all_reduce_population_count(mask)

        def lane_sum(v):
            for p in perms:
                bfly[...] = v
                v = v + plsc.load_gather(bfly, [p])
            return v

        def hist_zero():
            for j in range(NBINS // L):
                hist[pl.ds(j * L, L)] = zvec

        def hist_scan(r):
            run = zero
            less = zvec
            for j in range(NBINS // L):
                h = hist[pl.ds(j * L, L)]
                csg = plsc.cumsum(h) + run
                hist[pl.ds(j * L, L)] = csg
                run = csg[15]
                less = less + popc(csg < r)
            b = less[0]
            bm1 = jnp.maximum(b - 1, zero)
            prev = plsc.load_gather(hist, [jnp.full((L,), bm1, jnp.int32)])
            cbelow = jnp.where(b == 0, zero, prev[0])
            return b, cbelow

        kbufs = (kbuf0, kbuf1)
        for row in range(ROWS_PER_W):
            pltpu.sync_copy(x_hbm.at[wid * ROWS_PER_W + row], kbufs[row])

        res_v = zvec
        res_i = zvec

        for row in range(ROWS_PER_W):
            krow = kbufs[row]

            hist_zero()

            @plsc.parallel_loop(0, N_COLS // L, unroll=U)
            def pass_h1(j):
                bs = j * L
                b = krow[pl.ds(bs, L)]
                b = jnp.where(b == top, zero, b)
                m = lax.shift_right_arithmetic(b, 31)
                key = lax.bitwise_xor(b, lax.bitwise_or(m, top))
                krow[pl.ds(bs, L)] = key
                bin1 = lax.shift_right_logical(key, 24)
                plsc.addupdate_scatter(hist, [bin1], ones_v)
            r = jnp.int32(KTH)
            b1, cb1 = hist_scan(r)
            r = r - cb1

            prefix = b1
            for lvl in range(1, 4):
                hist_zero()
                hi_shift = 32 - 8 * lvl
                lo_shift = 24 - 8 * lvl

                @plsc.parallel_loop(0, N_COLS // L, unroll=U)
                def pass_h(j, hi_shift=hi_shift, lo_shift=lo_shift,
                           prefix=prefix):
                    bs = j * L
                    key = krow[pl.ds(bs, L)]
                    m = lax.shift_right_logical(key, hi_shift) == prefix
                    binv = lax.bitwise_and(
                        lax.shift_right_logical(key, lo_shift),
                        jnp.int32(0xFF))
                    plsc.addupdate_scatter(hist, [binv], ones_v, mask=m)
                bl, cbl = hist_scan(r)
                r = r - cbl
                prefix = lax.bitwise_or(
                    lax.shift_left(prefix, jnp.int32(8)), bl)

            v_ans = prefix

            @plsc.parallel_loop(0, N_COLS // L, unroll=U,
                                carry=(zvec, zvec))
            def pass_i(j, carry):
                cnt, pos = carry
                bs = j * L
                key = krow[pl.ds(bs, L)]
                match = key == v_ans
                mi = jnp.where(match, one, zero)
                csg = plsc.cumsum(mi) + cnt
                hit = jnp.logical_and(match, csg == r)
                pos = pos + jnp.where(hit, io + bs, zero)
                cnt = cnt + popc(match)
                return cnt, pos

            _, pos_acc = pass_i
            pos = lane_sum(pos_acc)

            lane = io == row
            res_v = jnp.where(lane, v_ans, res_v)
            res_i = jnp.where(lane, pos, res_i)

        inv = jnp.where(res_v < 0,
                        lax.bitwise_xor(res_v, top),
                        lax.bitwise_xor(res_v, jnp.int32(-1)))
        vstage[...] = inv
        istage[...] = res_i
        pltpu.sync_copy(vstage, vout_hbm.at[wid])
        pltpu.sync_copy(istage, iout_hbm.at[wid])

    return body(x_bits)


def kernel(x):
    xb = lax.bitcast_convert_type(x, jnp.int32)
    vbits, inds = _sc_kthvalue(xb)
    values = lax.bitcast_convert_type(
        vbits[:, :ROWS_PER_W].reshape(N_ROWS), jnp.float32)
    indices = inds[:, :ROWS_PER_W].reshape(N_ROWS)
    return values, indices.astype(jnp.int64)


# speedup vs baseline: 2.8469x; 1.0004x over previous
import functools

import jax
import jax.numpy as jnp
from jax import lax
from jax.experimental import pallas as pl
from jax.experimental.pallas import tpu as pltpu
from jax.experimental.pallas import tpu_sc as plsc

N_ROWS = 64
N_COLS = 8192
KTH = 256
NUM_CORES = 2
NUM_SUBCORES = 16
NW = NUM_CORES * NUM_SUBCORES
ROWS_PER_W = N_ROWS // NW
L = 16
U = 8
UL = U * L
NBINS = 256
TOP_I = -(2 ** 31)


def _sc_kthvalue(x_bits):
    mesh = plsc.VectorSubcoreMesh(
        core_axis_name="c", subcore_axis_name="s",
        num_cores=NUM_CORES, num_subcores=NUM_SUBCORES)

    @functools.partial(
        pl.kernel,
        out_type=(jax.ShapeDtypeStruct((NW, L), jnp.int32),
                  jax.ShapeDtypeStruct((NW, L), jnp.int32)),
        mesh=mesh,
        compiler_params=pltpu.CompilerParams(needs_layout_passes=False,
                                             skip_device_barrier=True),
        scratch_types=[
            pltpu.VMEM((N_COLS,), jnp.int32),
            pltpu.VMEM((N_COLS,), jnp.int32),
            pltpu.VMEM((NBINS,), jnp.int32),
            pltpu.VMEM((L,), jnp.int32),
            pltpu.VMEM((L,), jnp.int32),
            pltpu.VMEM((L,), jnp.int32),
        ],
    )
    def body(x_hbm, vout_hbm, iout_hbm, kbuf0, kbuf1, hist, bfly, vstage,
             istage):
        wid = lax.axis_index("s") * NUM_CORES + lax.axis_index("c")
        io = lax.iota(jnp.int32, L)
        perms = tuple(lax.bitwise_xor(io, jnp.int32(1 << p))
                      for p in range(3, -1, -1))
        one = jnp.int32(1)
        zero = jnp.int32(0)
        top = jnp.int32(TOP_I)
        zvec = jnp.zeros((L,), jnp.int32)
        ones_v = jnp.full((L,), 1, jnp.int32)

        def popc(mask):
            return plsc.all_reduce_population_count(mask)

        def lane_sum(v):
            for p in perms:
                bfly[...] = v
                v = v + plsc.load_gather(bfly, [p])
            return v

        def hist_zero():
            for j in range(NBINS // L):
                hist[pl.ds(j * L, L)] = zvec

        def hist_scan(r):
            run = zero
            less = zvec
            for j in range(NBINS // L):
                h = hist[pl.ds(j * L, L)]
                csg = plsc.cumsum(h) + run
                hist[pl.ds(j * L, L)] = csg
                run = csg[15]
                less = less + popc(csg < r)
            b = less[0]
            bm1 = jnp.maximum(b - 1, zero)
            prev = plsc.load_gather(hist, [jnp.full((L,), bm1, jnp.int32)])
            cbelow = jnp.where(b == 0, zero, prev[0])
            return b, cbelow

        kbufs = (kbuf0, kbuf1)
        for row in range(ROWS_PER_W):
            pltpu.sync_copy(x_hbm.at[wid * ROWS_PER_W + row], kbufs[row])

        res_v = zvec
        res_i = zvec

        for row in range(ROWS_PER_W):
            krow = kbufs[row]

            hist_zero()

            @plsc.parallel_loop(0, N_COLS // L, unroll=U)
            def pass_h1(j):
                bs = j * L
                b = krow[pl.ds(bs, L)]
                b = jnp.where(b == top, zero, b)
                m = lax.shift_right_arithmetic(b, 31)
                key = lax.bitwise_xor(b, lax.bitwise_or(m, top))
                krow[pl.ds(bs, L)] = key
                bin1 = lax.shift_right_logical(key, 24)
                plsc.addupdate_scatter(hist, [bin1], ones_v)
            r = jnp.int32(KTH)
            b1, cb1 = hist_scan(r)
            r = r - cb1

            prefix = b1
            for lvl in range(1, 4):
                hist_zero()
                hi_shift = 32 - 8 * lvl
                lo_shift = 24 - 8 * lvl

                @plsc.parallel_loop(0, N_COLS // L, unroll=U)
                def pass_h(j, hi_shift=hi_shift, lo_shift=lo_shift,
                           prefix=prefix):
                    bs = j * L
                    key = krow[pl.ds(bs, L)]
                    m = lax.shift_right_logical(key, hi_shift) == prefix
                    binv = lax.bitwise_and(
                        lax.shift_right_logical(key, lo_shift),
                        jnp.int32(0xFF))
                    plsc.addupdate_scatter(hist, [binv], ones_v, mask=m)
                bl, cbl = hist_scan(r)
                r = r - cbl
                prefix = lax.bitwise_or(
                    lax.shift_left(prefix, jnp.int32(8)), bl)

            v_ans = prefix

            @plsc.parallel_loop(0, N_COLS // L, unroll=U,
                                carry=(zvec, zvec))
            def pass_i(j, carry):
                cnt, pos = carry
                bs = j * L
                key = krow[pl.ds(bs, L)]
                match = key == v_ans
                mi = jnp.where(match, one, zero)
                csg = plsc.cumsum(mi) + cnt
                hit = jnp.logical_and(match, csg == r)
                pos = pos + jnp.where(hit, io + bs, zero)
                cnt = cnt + popc(match)
                return cnt, pos

            _, pos_acc = pass_i
            pos = lane_sum(pos_acc)

            lane = io == row
            res_v = jnp.where(lane, v_ans, res_v)
            res_i = jnp.where(lane, pos, res_i)

        inv = jnp.where(res_v < 0,
                        lax.bitwise_xor(res_v, top),
                        lax.bitwise_xor(res_v, jnp.int32(-1)))
        vstage[...] = inv
        istage[...] = res_i
        pltpu.sync_copy(vstage, vout_hbm.at[wid])
        pltpu.sync_copy(istage, iout_hbm.at[wid])

    return body(x_bits)


def kernel(x):
    xb = lax.bitcast_convert_type(x, jnp.int32)
    vbits, inds = _sc_kthvalue(xb)
    values = lax.bitcast_convert_type(
        vbits[:, :ROWS_PER_W].reshape(N_ROWS), jnp.float32)
    indices = inds[:, :ROWS_PER_W].reshape(N_ROWS)
    return values, indices.astype(jnp.int64)


# speedup vs baseline: 2.9644x; 1.0413x over previous
import functools

import jax
import jax.numpy as jnp
from jax import lax
from jax.experimental import pallas as pl
from jax.experimental.pallas import tpu as pltpu
from jax.experimental.pallas import tpu_sc as plsc

N_ROWS = 64
N_COLS = 8192
KTH = 256
NUM_CORES = 2
NUM_SUBCORES = 16
NW = NUM_CORES * NUM_SUBCORES
ROWS_PER_W = N_ROWS // NW
L = 16
U = 8
UL = U * L
NBINS = 256
TOP_I = -(2 ** 31)


def _sc_kthvalue(x_bits):
    mesh = plsc.VectorSubcoreMesh(
        core_axis_name="c", subcore_axis_name="s",
        num_cores=NUM_CORES, num_subcores=NUM_SUBCORES)

    @functools.partial(
        pl.kernel,
        out_type=(jax.ShapeDtypeStruct((NW, L), jnp.int32),
                  jax.ShapeDtypeStruct((NW, L), jnp.int32)),
        mesh=mesh,
        compiler_params=pltpu.CompilerParams(needs_layout_passes=False),
        scratch_types=[
            pltpu.VMEM((N_COLS,), jnp.int32),
            pltpu.VMEM((N_COLS,), jnp.int32),
            pltpu.VMEM((NBINS,), jnp.int32),
            pltpu.VMEM((NBINS,), jnp.int32),
            pltpu.VMEM((L,), jnp.int32),
            pltpu.VMEM((L,), jnp.int32),
            pltpu.VMEM((L,), jnp.int32),
        ],
    )
    def body(x_hbm, vout_hbm, iout_hbm, kbuf0, kbuf1, hist, posa, bfly,
             vstage, istage):
        wid = lax.axis_index("s") * NUM_CORES + lax.axis_index("c")
        io = lax.iota(jnp.int32, L)
        perms = tuple(lax.bitwise_xor(io, jnp.int32(1 << p))
                      for p in range(3, -1, -1))
        one = jnp.int32(1)
        zero = jnp.int32(0)
        top = jnp.int32(TOP_I)
        zvec = jnp.zeros((L,), jnp.int32)
        ones_v = jnp.full((L,), 1, jnp.int32)

        def popc(mask):
            return plsc.all_reduce_population_count(mask)

        def lane_sum(v):
            for p in perms:
                bfly[...] = v
                v = v + plsc.load_gather(bfly, [p])
            return v

        def hist_zero():
            for j in range(NBINS // L):
                hist[pl.ds(j * L, L)] = zvec

        def posa_zero():
            for j in range(NBINS // L):
                posa[pl.ds(j * L, L)] = zvec

        def hist_scan(r):
            run = zero
            less = zvec
            for j in range(NBINS // L):
                h = hist[pl.ds(j * L, L)]
                csg = plsc.cumsum(h) + run
                hist[pl.ds(j * L, L)] = csg
                run = csg[15]
                less = less + popc(csg < r)
            b = less[0]
            bm1 = jnp.maximum(b - 1, zero)
            prev = plsc.load_gather(hist, [jnp.full((L,), bm1, jnp.int32)])
            cbelow = jnp.where(b == 0, zero, prev[0])
            cum_b = plsc.load_gather(hist, [jnp.full((L,), b, jnp.int32)])
            nbin = cum_b[0] - cbelow
            return b, cbelow, nbin

        kbufs = (kbuf0, kbuf1)
        for row in range(ROWS_PER_W):
            pltpu.sync_copy(x_hbm.at[wid * ROWS_PER_W + row], kbufs[row])

        res_v = zvec
        res_i = zvec

        for row in range(ROWS_PER_W):
            krow = kbufs[row]

            hist_zero()

            @plsc.parallel_loop(0, N_COLS // L, unroll=U)
            def pass_h1(j):
                bs = j * L
                b = krow[pl.ds(bs, L)]
                b = jnp.where(b == top, zero, b)
                m = lax.shift_right_arithmetic(b, 31)
                key = lax.bitwise_xor(b, lax.bitwise_or(m, top))
                krow[pl.ds(bs, L)] = key
                bin1 = lax.shift_right_logical(key, 24)
                plsc.addupdate_scatter(hist, [bin1], ones_v)
            r = jnp.int32(KTH)
            b1, cb1, _ = hist_scan(r)
            r = r - cb1

            hist_zero()

            @plsc.parallel_loop(0, N_COLS // L, unroll=U)
            def pass_h2(j):
                bs = j * L
                key = krow[pl.ds(bs, L)]
                m = lax.shift_right_logical(key, 24) == b1
                binv = lax.bitwise_and(
                    lax.shift_right_logical(key, 16), jnp.int32(0xFF))
                plsc.addupdate_scatter(hist, [binv], ones_v, mask=m)
            b2, cb2, _ = hist_scan(r)
            r = r - cb2
            p16 = lax.bitwise_or(lax.shift_left(b1, jnp.int32(8)), b2)

            hist_zero()
            posa_zero()

            @plsc.parallel_loop(0, N_COLS // L, unroll=U)
            def pass_h3(j):
                bs = j * L
                key = krow[pl.ds(bs, L)]
                m = lax.shift_right_logical(key, 16) == p16
                binv = lax.bitwise_and(
                    lax.shift_right_logical(key, 8), jnp.int32(0xFF))
                plsc.addupdate_scatter(hist, [binv], ones_v, mask=m)
                plsc.addupdate_scatter(posa, [binv], io + bs, mask=m)
            b3, cb3, n3 = hist_scan(r)
            r = r - cb3
            p24 = lax.bitwise_or(lax.shift_left(p16, jnp.int32(8)), b3)

            def fast3(_):
                idxv = plsc.load_gather(
                    posa, [jnp.full((L,), b3, jnp.int32)])
                keyv = plsc.load_gather(krow, [idxv])
                return keyv, idxv

            def slow3(_):
                hist_zero()
                posa_zero()

                @plsc.parallel_loop(0, N_COLS // L, unroll=U)
                def pass_h4(j):
                    bs = j * L
                    key = krow[pl.ds(bs, L)]
                    m = lax.shift_right_logical(key, 8) == p24
                    binv = lax.bitwise_and(key, jnp.int32(0xFF))
                    plsc.addupdate_scatter(hist, [binv], ones_v, mask=m)
                    plsc.addupdate_scatter(posa, [binv], io + bs, mask=m)
                b4, cb4, n4 = hist_scan(r)
                r4 = r - cb4
                v_ans = lax.bitwise_or(lax.shift_left(p24, jnp.int32(8)), b4)

                def fast4(_):
                    return plsc.load_gather(
                        posa, [jnp.full((L,), b4, jnp.int32)])

                def slow4(_):
                    @plsc.parallel_loop(0, N_COLS // L, unroll=U,
                                        carry=(zvec, zvec))
                    def pass_i(j, carry):
                        cnt, pos = carry
                        bs = j * L
                        key = krow[pl.ds(bs, L)]
                        match = key == v_ans
                        mi = jnp.where(match, one, zero)
                        csg = plsc.cumsum(mi) + cnt
                        hit = jnp.logical_and(match, csg == r4)
                        pos = pos + jnp.where(hit, io + bs, zero)
                        cnt = cnt + popc(match)
                        return cnt, pos

                    _, pos_acc = pass_i
                    return lane_sum(pos_acc)

                posv = lax.cond(n4 == 1, fast4, slow4, zero)
                return zvec + v_ans, posv

            key_vec, pos_vec = lax.cond(n3 == 1, fast3, slow3, zero)

            lane = io == row
            res_v = jnp.where(lane, key_vec, res_v)
            res_i = jnp.where(lane, pos_vec, res_i)

        inv = jnp.where(res_v < 0,
                        lax.bitwise_xor(res_v, top),
                        lax.bitwise_xor(res_v, jnp.int32(-1)))
        vstage[...] = inv
        istage[...] = res_i
        pltpu.sync_copy(vstage, vout_hbm.at[wid])
        pltpu.sync_copy(istage, iout_hbm.at[wid])

    return body(x_bits)


def kernel(x):
    xb = lax.bitcast_convert_type(x, jnp.int32)
    vbits, inds = _sc_kthvalue(xb)
    values = lax.bitcast_convert_type(
        vbits[:, :ROWS_PER_W].reshape(N_ROWS), jnp.float32)
    indices = inds[:, :ROWS_PER_W].reshape(N_ROWS)
    return values, indices.astype(jnp.int64)


# speedup vs baseline: 3.0221x; 1.0194x over previous
import functools

import jax
import jax.numpy as jnp
from jax import lax
from jax.experimental import pallas as pl
from jax.experimental.pallas import tpu as pltpu
from jax.experimental.pallas import tpu_sc as plsc

N_ROWS = 64
N_COLS = 8192
KTH = 256
NUM_CORES = 2
NUM_SUBCORES = 16
NW = NUM_CORES * NUM_SUBCORES
ROWS_PER_W = N_ROWS // NW
L = 16
U = 4
UL = U * L
NBINS = 256
TOP_I = -(2 ** 31)


def _sc_kthvalue(x_bits):
    mesh = plsc.VectorSubcoreMesh(
        core_axis_name="c", subcore_axis_name="s",
        num_cores=NUM_CORES, num_subcores=NUM_SUBCORES)

    @functools.partial(
        pl.kernel,
        out_type=(jax.ShapeDtypeStruct((NW, L), jnp.int32),
                  jax.ShapeDtypeStruct((NW, L), jnp.int32)),
        mesh=mesh,
        compiler_params=pltpu.CompilerParams(needs_layout_passes=False),
        scratch_types=[
            pltpu.VMEM((N_COLS,), jnp.int32),
            pltpu.VMEM((N_COLS,), jnp.int32),
            pltpu.VMEM((NBINS,), jnp.int32),
            pltpu.VMEM((NBINS,), jnp.int32),
            pltpu.VMEM((NBINS,), jnp.int32),
            pltpu.VMEM((NBINS,), jnp.int32),
            pltpu.VMEM((L,), jnp.int32),
            pltpu.VMEM((L,), jnp.int32),
            pltpu.VMEM((L,), jnp.int32),
        ],
    )
    def body(x_hbm, vout_hbm, iout_hbm, kbuf0, kbuf1, histA, histB, posaA,
             posaB, bfly, vstage, istage):
        wid = lax.axis_index("s") * NUM_CORES + lax.axis_index("c")
        io = lax.iota(jnp.int32, L)
        perms = tuple(lax.bitwise_xor(io, jnp.int32(1 << p))
                      for p in range(3, -1, -1))
        one = jnp.int32(1)
        zero = jnp.int32(0)
        top = jnp.int32(TOP_I)
        zvec = jnp.zeros((L,), jnp.int32)
        ones_v = jnp.full((L,), 1, jnp.int32)

        kbufs = (kbuf0, kbuf1)
        hists = (histA, histB)
        posas = (posaA, posaB)

        def popc(mask):
            return plsc.all_reduce_population_count(mask)

        def lane_sum(v):
            for p in perms:
                bfly[...] = v
                v = v + plsc.load_gather(bfly, [p])
            return v

        def zero_bins(ref):
            for j in range(NBINS // L):
                ref[pl.ds(j * L, L)] = zvec

        def hist_scan(hist, r):
            run = zero
            less = zvec
            for j in range(NBINS // L):
                h = hist[pl.ds(j * L, L)]
                csg = plsc.cumsum(h) + run
                hist[pl.ds(j * L, L)] = csg
                run = csg[15]
                less = less + popc(csg < r)
            b = less[0]
            bm1 = jnp.maximum(b - 1, zero)
            prev = plsc.load_gather(hist, [jnp.full((L,), bm1, jnp.int32)])
            cbelow = jnp.where(b == 0, zero, prev[0])
            cum_b = plsc.load_gather(hist, [jnp.full((L,), b, jnp.int32)])
            nbin = cum_b[0] - cbelow
            return b, cbelow, nbin

        for row in range(ROWS_PER_W):
            pltpu.sync_copy(x_hbm.at[wid * ROWS_PER_W + row], kbufs[row])

        zero_bins(histA)
        zero_bins(histB)

        @plsc.parallel_loop(0, N_COLS // L, unroll=U)
        def pass_h1(j):
            bs = j * L
            for row in range(ROWS_PER_W):
                krow = kbufs[row]
                b = krow[pl.ds(bs, L)]
                b = jnp.where(b == top, zero, b)
                m = lax.shift_right_arithmetic(b, 31)
                key = lax.bitwise_xor(b, lax.bitwise_or(m, top))
                krow[pl.ds(bs, L)] = key
                bin1 = lax.shift_right_logical(key, 24)
                plsc.addupdate_scatter(hists[row], [bin1], ones_v)

        rs, b1s = [], []
        for row in range(ROWS_PER_W):
            b1, cb1, _ = hist_scan(hists[row], jnp.int32(KTH))
            b1s.append(b1)
            rs.append(jnp.int32(KTH) - cb1)

        zero_bins(histA)
        zero_bins(histB)

        @plsc.parallel_loop(0, N_COLS // L, unroll=U)
        def pass_h2(j):
            bs = j * L
            for row in range(ROWS_PER_W):
                key = kbufs[row][pl.ds(bs, L)]
                m = lax.shift_right_logical(key, 24) == b1s[row]
                binv = lax.bitwise_and(
                    lax.shift_right_logical(key, 16), jnp.int32(0xFF))
                plsc.addupdate_scatter(hists[row], [binv], ones_v, mask=m)

        p16s = []
        for row in range(ROWS_PER_W):
            b2, cb2, _ = hist_scan(hists[row], rs[row])
            rs[row] = rs[row] - cb2
            p16s.append(lax.bitwise_or(
                lax.shift_left(b1s[row], jnp.int32(8)), b2))

        zero_bins(histA)
        zero_bins(histB)
        zero_bins(posaA)
        zero_bins(posaB)

        @plsc.parallel_loop(0, N_COLS // L, unroll=U)
        def pass_h3(j):
            bs = j * L
            for row in range(ROWS_PER_W):
                key = kbufs[row][pl.ds(bs, L)]
                m = lax.shift_right_logical(key, 16) == p16s[row]
                binv = lax.bitwise_and(
                    lax.shift_right_logical(key, 8), jnp.int32(0xFF))
                plsc.addupdate_scatter(hists[row], [binv], ones_v, mask=m)
                plsc.addupdate_scatter(posas[row], [binv], io + bs, mask=m)

        res_v = zvec
        res_i = zvec
        for row in range(ROWS_PER_W):
            krow = kbufs[row]
            hist = hists[row]
            posa = posas[row]
            b3, cb3, n3 = hist_scan(hist, rs[row])
            r = rs[row] - cb3
            p24 = lax.bitwise_or(lax.shift_left(p16s[row], jnp.int32(8)), b3)

            def fast3(_, posa=posa, krow=krow, b3=b3):
                idxv = plsc.load_gather(
                    posa, [jnp.full((L,), b3, jnp.int32)])
                keyv = plsc.load_gather(krow, [idxv])
                return keyv, idxv

            def slow3(_, krow=krow, hist=hist, posa=posa, p24=p24, r=r):
                zero_bins(hist)
                zero_bins(posa)

                @plsc.parallel_loop(0, N_COLS // L, unroll=U)
                def pass_h4(j):
                    bs = j * L
                    key = krow[pl.ds(bs, L)]
                    m = lax.shift_right_logical(key, 8) == p24
                    binv = lax.bitwise_and(key, jnp.int32(0xFF))
                    plsc.addupdate_scatter(hist, [binv], ones_v, mask=m)
                    plsc.addupdate_scatter(posa, [binv], io + bs, mask=m)
                b4, cb4, n4 = hist_scan(hist, r)
                r4 = r - cb4
                v_ans = lax.bitwise_or(lax.shift_left(p24, jnp.int32(8)), b4)

                def fast4(_):
                    return plsc.load_gather(
                        posa, [jnp.full((L,), b4, jnp.int32)])

                def slow4(_):
                    @plsc.parallel_loop(0, N_COLS // L, unroll=U,
                                        carry=(zvec, zvec))
                    def pass_i(j, carry):
                        cnt, pos = carry
                        bs = j * L
                        key = krow[pl.ds(bs, L)]
                        match = key == v_ans
                        mi = jnp.where(match, one, zero)
                        csg = plsc.cumsum(mi) + cnt
                        hit = jnp.logical_and(match, csg == r4)
                        pos = pos + jnp.where(hit, io + bs, zero)
                        cnt = cnt + popc(match)
                        return cnt, pos

                    _, pos_acc = pass_i
                    return lane_sum(pos_acc)

                posv = lax.cond(n4 == 1, fast4, slow4, zero)
                return zvec + v_ans, posv

            key_vec, pos_vec = lax.cond(n3 == 1, fast3, slow3, zero)

            lane = io == row
            res_v = jnp.where(lane, key_vec, res_v)
            res_i = jnp.where(lane, pos_vec, res_i)

        inv = jnp.where(res_v < 0,
                        lax.bitwise_xor(res_v, top),
                        lax.bitwise_xor(res_v, jnp.int32(-1)))
        vstage[...] = inv
        istage[...] = res_i
        pltpu.sync_copy(vstage, vout_hbm.at[wid])
        pltpu.sync_copy(istage, iout_hbm.at[wid])

    return body(x_bits)


def kernel(x):
    xb = lax.bitcast_convert_type(x, jnp.int32)
    vbits, inds = _sc_kthvalue(xb)
    values = lax.bitcast_convert_type(
        vbits[:, :ROWS_PER_W].reshape(N_ROWS), jnp.float32)
    indices = inds[:, :ROWS_PER_W].reshape(N_ROWS)
    return values, indices.astype(jnp.int64)
